# Initial kernel scaffold; baseline (speedup 1.0000x reference)
import functools
import math

import jax
import jax.numpy as jnp
from jax import lax
from jax.experimental import pallas as pl
from jax.experimental.pallas import tpu as pltpu
from jax.experimental.pallas import tpu_sc as plsc

EMB = 64
SCALE = math.sqrt(EMB)

NUM_CORES = 2
NUM_SUBCORES = 16
NUM_WORKERS = NUM_CORES * NUM_SUBCORES

CHUNK = 512


def _scale_body(t_ref, o_ref):
    o_ref[...] = t_ref[...] * SCALE


@jax.jit
def _scale_table(table):
    vocab, emb = table.shape
    block = 5000
    assert vocab % block == 0
    return pl.pallas_call(
        _scale_body,
        out_shape=jax.ShapeDtypeStruct(table.shape, table.dtype),
        grid=(vocab // block,),
        in_specs=[pl.BlockSpec((block, emb), lambda i: (i, 0))],
        out_specs=pl.BlockSpec((block, emb), lambda i: (i, 0)),
    )(table)


def _gather_body(table_hbm, idx_hbm, out_hbm, idx_v, rows_v, gsem, ssem):
    wid = lax.axis_index("s") * NUM_CORES + lax.axis_index("c")
    n_total = idx_hbm.shape[0]
    bpw = n_total // NUM_WORKERS
    base = wid * bpw
    pltpu.sync_copy(idx_hbm.at[pl.ds(base, bpw)], idx_v)

    @pl.loop(0, bpw // CHUNK)
    def _chunk(g):
        off = g * CHUNK
        pltpu.async_copy(
            table_hbm.at[idx_v.at[pl.ds(off, CHUNK)]], rows_v, gsem
        ).wait()
        pltpu.async_copy(rows_v, out_hbm.at[pl.ds(base + off, CHUNK)], ssem).wait()


def _make_gather(n_total):
    mesh = plsc.VectorSubcoreMesh(core_axis_name="c", subcore_axis_name="s")
    bpw = n_total // NUM_WORKERS
    return pl.kernel(
        _gather_body,
        out_type=jax.ShapeDtypeStruct((n_total, EMB), jnp.float32),
        mesh=mesh,
        scratch_types=[
            pltpu.VMEM((bpw,), jnp.int32),
            pltpu.VMEM((CHUNK, EMB), jnp.float32),
            pltpu.SemaphoreType.DMA,
            pltpu.SemaphoreType.DMA,
        ],
    )


def kernel(x, table):
    b, t = x.shape
    n_total = b * t
    scaled = _scale_table(table)
    flat = x.reshape(n_total)
    out = _make_gather(n_total)(scaled, flat)
    return out.reshape(b, t, EMB)


# speedup vs baseline: 3.7877x; 3.7877x over previous
import functools
import math

import jax
import jax.numpy as jnp
from jax import lax
from jax.experimental import pallas as pl
from jax.experimental.pallas import tpu as pltpu
from jax.experimental.pallas import tpu_sc as plsc

EMB = 64
SCALE = math.sqrt(EMB)

NUM_CORES = 2
NUM_SUBCORES = 16
NUM_WORKERS = NUM_CORES * NUM_SUBCORES

CHUNK = 512


def _scale_body(t_ref, o_ref):
    o_ref[...] = t_ref[...] * SCALE


@jax.jit
def _scale_table(table):
    vocab, emb = table.shape
    block = 5000
    assert vocab % block == 0
    return pl.pallas_call(
        _scale_body,
        out_shape=jax.ShapeDtypeStruct(table.shape, table.dtype),
        grid=(vocab // block,),
        in_specs=[pl.BlockSpec((block, emb), lambda i: (i, 0))],
        out_specs=pl.BlockSpec((block, emb), lambda i: (i, 0)),
    )(table)


def _gather_body(table_hbm, idx_hbm, out_hbm, idx_v, rows_v, gsem, ssem):
    wid = lax.axis_index("s") * NUM_CORES + lax.axis_index("c")
    n_total = idx_hbm.shape[0]
    bpw = n_total // NUM_WORKERS
    base = wid * bpw
    pltpu.sync_copy(idx_hbm.at[pl.ds(base, bpw)], idx_v)

    @pl.loop(0, bpw // CHUNK)
    def _chunk(g):
        off = g * CHUNK
        pltpu.async_copy(
            table_hbm.at[idx_v.at[pl.ds(off, CHUNK)]], rows_v, gsem
        ).wait()
        pltpu.async_copy(rows_v, out_hbm.at[pl.ds(base + off, CHUNK)], ssem).wait()


def _make_gather(n_total):
    mesh = plsc.VectorSubcoreMesh(core_axis_name="c", subcore_axis_name="s")
    bpw = n_total // NUM_WORKERS
    return pl.kernel(
        _gather_body,
        out_type=jax.ShapeDtypeStruct((n_total, EMB), jnp.float32),
        mesh=mesh,
        scratch_types=[
            pltpu.VMEM((bpw,), jnp.int32),
            pltpu.VMEM((CHUNK, EMB), jnp.float32),
            pltpu.SemaphoreType.DMA,
            pltpu.SemaphoreType.DMA,
        ],
        compiler_params=pltpu.CompilerParams(use_tc_tiling_on_sc=False),
    )


def kernel(x, table):
    b, t = x.shape
    n_total = b * t
    scaled = _scale_table(table)
    flat = x.reshape(n_total)
    out = _make_gather(n_total)(scaled, flat)
    return out.reshape(b, t, EMB)


# speedup vs baseline: 3.9423x; 1.0408x over previous
import functools
import math

import jax
import jax.numpy as jnp
from jax import lax
from jax.experimental import pallas as pl
from jax.experimental.pallas import tpu as pltpu
from jax.experimental.pallas import tpu_sc as plsc

EMB = 64
SCALE = math.sqrt(EMB)

NUM_CORES = 2
NUM_SUBCORES = 16
NUM_WORKERS = NUM_CORES * NUM_SUBCORES

CHUNK = 512


def _scale_body(t_ref, o_ref):
    o_ref[...] = t_ref[...] * SCALE


@jax.jit
def _scale_table(table):
    vocab, emb = table.shape
    block = 5000
    assert vocab % block == 0
    return pl.pallas_call(
        _scale_body,
        out_shape=jax.ShapeDtypeStruct(table.shape, table.dtype),
        grid=(vocab // block,),
        in_specs=[pl.BlockSpec((block, emb), lambda i: (i, 0))],
        out_specs=pl.BlockSpec((block, emb), lambda i: (i, 0)),
    )(table)


def _gather_body(table_hbm, idx_hbm, out_hbm, idx_v, rows0, rows1, gs0, gs1,
                 ss0, ss1):
    wid = lax.axis_index("s") * NUM_CORES + lax.axis_index("c")
    n_total = idx_hbm.shape[0]
    bpw = n_total // NUM_WORKERS
    nchunk = bpw // CHUNK
    npair = nchunk // 2
    base = wid * bpw
    pltpu.sync_copy(idx_hbm.at[pl.ds(base, bpw)], idx_v)

    rows = (rows0, rows1)
    gsems = (gs0, gs1)
    ssems = (ss0, ss1)

    def start_gather(g, b):
        off = g * CHUNK
        pltpu.async_copy(
            table_hbm.at[idx_v.at[pl.ds(off, CHUNK)]], rows[b], gsems[b]
        )

    def start_store(g, b):
        off = g * CHUNK
        pltpu.async_copy(rows[b], out_hbm.at[pl.ds(base + off, CHUNK)], ssems[b])

    def wait_gather(b):
        pltpu.make_async_copy(
            table_hbm.at[pl.ds(0, CHUNK)], rows[b], gsems[b]
        ).wait()

    def wait_store(b):
        pltpu.make_async_copy(
            rows[b], out_hbm.at[pl.ds(base, CHUNK)], ssems[b]
        ).wait()

    start_gather(0, 0)

    @pl.loop(0, npair)
    def _pair(i):
        g0 = 2 * i
        @pl.when(i > 0)
        def _():
            wait_store(1)

        start_gather(g0 + 1, 1)

        wait_gather(0)
        start_store(g0, 0)

        wait_store(0)

        @pl.when(i + 1 < npair)
        def _():
            start_gather(g0 + 2, 0)

        wait_gather(1)
        start_store(g0 + 1, 1)

    wait_store(1)


def _make_gather(n_total):
    mesh = plsc.VectorSubcoreMesh(core_axis_name="c", subcore_axis_name="s")
    bpw = n_total // NUM_WORKERS
    return pl.kernel(
        _gather_body,
        out_type=jax.ShapeDtypeStruct((n_total, EMB), jnp.float32),
        mesh=mesh,
        scratch_types=[
            pltpu.VMEM((bpw,), jnp.int32),
            pltpu.VMEM((CHUNK, EMB), jnp.float32),
            pltpu.VMEM((CHUNK, EMB), jnp.float32),
            pltpu.SemaphoreType.DMA,
            pltpu.SemaphoreType.DMA,
            pltpu.SemaphoreType.DMA,
            pltpu.SemaphoreType.DMA,
        ],
        compiler_params=pltpu.CompilerParams(use_tc_tiling_on_sc=False),
    )


def kernel(x, table):
    b, t = x.shape
    n_total = b * t
    scaled = _scale_table(table)
    flat = x.reshape(n_total)
    out = _make_gather(n_total)(scaled, flat)
    return out.reshape(b, t, EMB)
